# Initial kernel scaffold; baseline (speedup 1.0000x reference)
import jax
import jax.numpy as jnp
from jax.experimental import pallas as pl


def kernel(x, params):
    raise NotImplementedError("write your pallas kernel here")


# speedup vs baseline: 2.5631x; 2.5631x over previous
import functools

import jax
import jax.numpy as jnp
import numpy as np
from jax.experimental import pallas as pl
from jax.experimental.pallas import tpu as pltpu

EPS5 = 1e-5
RS = 1.0 / np.sqrt(1.0 + EPS5)
BIGF = 3.0e38
SAMP4_IDX = jnp.asarray(
    np.random.default_rng(42).choice(1024, size=120, replace=False),
    dtype=jnp.int32)


def _discard(d2, iota, n):
  m = jnp.min(d2, axis=1, keepdims=True)
  cand = jnp.where(d2 <= m, iota, n)
  idx = jnp.min(cand, axis=1, keepdims=True)
  return jnp.where(cand == idx, BIGF, d2)


def _extract(d2, iota, n):
  m = jnp.min(d2, axis=1, keepdims=True)
  cand = jnp.where(d2 <= m, iota, n)
  idx = jnp.min(cand, axis=1, keepdims=True)
  oh = (cand == idx).astype(jnp.float32)
  return oh, jnp.where(cand == idx, BIGF, d2)


def _mm(a, b):
  return jax.lax.dot(a, b, precision=jax.lax.Precision.HIGHEST,
                     preferred_element_type=jnp.float32)


def _mmbf(a, b):
  return jax.lax.dot(a.astype(jnp.bfloat16), b.astype(jnp.bfloat16),
                     preferred_element_type=jnp.float32)


def _bf(a):
  return a.astype(jnp.bfloat16).astype(jnp.float32)


def _layer_body(K, D, P, N, cmid, cxin, cout, dm,
                rep_ref, pts_ref, ptsT_ref, fts_ref,
                WinT, bin_, sin_, tin_,
                BD1, b1, s1, t1, BD2, b2, s2, t2,
                Wx, bx, Wxd1, bxd1, Wxd2, bxd2,
                Wdf, Wde, WpTf, WpTe, bdf, bde, ssep, tsep,
                out_ref):
  rep = rep_ref[0]
  pts = pts_ref[0]
  ptsT = ptsT_ref[0]
  fts = fts_ref[0]

  fd = jax.nn.relu(_mmbf(fts, WinT[...]) + bin_[...]) * sin_[...] + tin_[...]

  r2 = jnp.sum(rep * rep, axis=1, keepdims=True)
  p2 = jnp.sum(ptsT * ptsT, axis=0, keepdims=True)
  d2 = r2 - 2.0 * _mmbf(rep, ptsT) + p2

  iota = jax.lax.broadcasted_iota(jnp.int32, (P, N), 1)

  d2 = _discard(d2, iota, N)
  ptsg = []
  ftsg = []
  for kk in range(K):
    oh, d2 = _extract(d2, iota, N)
    ptsg.append(_mm(oh, pts))
    ftsg.append(_mm(oh, fd))
    if kk < K - 1 and D > 1:
      if D <= 3:
        for _ in range(D - 1):
          d2 = _discard(d2, iota, N)
      else:
        d2 = jax.lax.fori_loop(
            0, D - 1, lambda i, a: _discard(a, iota, N), d2)

  pl_flat = jnp.concatenate(ptsg, axis=1) - jnp.concatenate([rep] * K, axis=1)

  f = jax.nn.relu(_mmbf(pl_flat, BD1[...]) + b1[...]) * s1[...] + t1[...]
  f = jax.nn.relu(_mmbf(f, BD2[...]) + b2[...]) * s2[...] + t2[...]

  X = jax.nn.relu(_mmbf(pl_flat, Wx[...]) + bx[...])
  X = jax.nn.relu(_mmbf(X, Wxd1[...]) + bxd1[...])
  X = _mmbf(X, Wxd2[...]) + bxd2[...]

  Xbf = _bf(X)
  fbf = _bf(f)
  ftsgbf = [_bf(t) for t in ftsg]
  Wdf_ = _bf(Wdf[...])
  Wde_ = _bf(Wde[...])
  bdf_ = bdf[...]
  bde_ = bde[...]
  acc = jnp.zeros((P, cout), jnp.float32)
  for m in range(dm):
    ymf = jnp.zeros((P, cmid), jnp.float32)
    yme = jnp.zeros((P, cxin), jnp.float32)
    for i in range(K):
      fXf = jnp.zeros((P, cmid), jnp.float32)
      fXe = jnp.zeros((P, cxin), jnp.float32)
      for j in range(K):
        xij = Xbf[:, i * K + j:i * K + j + 1]
        fXf = fXf + xij * fbf[:, j * cmid:(j + 1) * cmid]
        fXe = fXe + xij * ftsgbf[j]
      ymf = ymf + _bf(fXf) * Wdf_[m * K + i:m * K + i + 1, :]
      yme = yme + _bf(fXe) * Wde_[m * K + i:m * K + i + 1, :]
    acc = acc + _mmbf(ymf + bdf_[m:m + 1, :], WpTf[...][m * cmid:(m + 1) * cmid, :])
    acc = acc + _mmbf(yme + bde_[m:m + 1, :], WpTe[...][m * cxin:(m + 1) * cxin, :])
  out = jax.nn.relu(acc) * ssep[...] + tsep[...]
  out_ref[0] = out


def _row(v):
  return v.reshape(1, -1)


def _prep_dense(p):
  if 'gamma' in p:
    return p['W'].T, _row(p['b']), _row(p['gamma'] * RS), _row(p['beta'])
  return p['W'].T, _row(p['b']), None, None


def _prep_layer(p, K, dm, cin, cout):
  cmid = cout // 4
  cxin = cout // 2
  xc = p['xconv']
  WinT, bin_, sin_, tin_ = _prep_dense(p['dense_in'])
  eyeK = jnp.eye(K, dtype=jnp.float32)
  BD1 = jnp.kron(eyeK, xc['d1']['W'].T)
  b1 = _row(jnp.tile(xc['d1']['b'], K))
  s1 = _row(jnp.tile(xc['d1']['gamma'] * RS, K))
  t1 = _row(jnp.tile(xc['d1']['beta'], K))
  BD2 = jnp.kron(eyeK, xc['d2']['W'].T)
  b2 = _row(jnp.tile(xc['d2']['b'], K))
  s2 = _row(jnp.tile(xc['d2']['gamma'] * RS, K))
  t2 = _row(jnp.tile(xc['d2']['beta'], K))
  Wx = jnp.transpose(xc['xc_W'][:, :, 0, :], (2, 1, 0)).reshape(3 * K, K * K)
  bx = _row(xc['xc_b'])
  Wxd1 = xc['xd1']['W'].T
  bxd1 = _row(xc['xd1']['b'])
  Wxd2 = xc['xd2']['W'].T
  bxd2 = _row(xc['xd2']['b'])
  sep = xc['sep']
  cc = cmid + cxin
  Wd3 = sep['Wd'][:, 0, 0, :].reshape(cc, dm, K)
  Wdf = jnp.transpose(Wd3[:cmid], (1, 2, 0)).reshape(dm * K, cmid)
  Wde = jnp.transpose(Wd3[cmid:], (1, 2, 0)).reshape(dm * K, cxin)
  Wp2 = sep['Wp'][:, :, 0, 0]
  Wp3 = Wp2.reshape(cout, cc, dm)
  WpTf = jnp.transpose(Wp3[:, :cmid, :], (2, 1, 0)).reshape(dm * cmid, cout)
  WpTe = jnp.transpose(Wp3[:, cmid:, :], (2, 1, 0)).reshape(dm * cxin, cout)
  bd2 = sep['bd'].reshape(cc, dm)
  bdf = bd2[:cmid].T
  bde = bd2[cmid:].T
  ssep = _row(sep['gamma'] * RS)
  tsep = _row(sep['beta'])
  return (WinT, bin_, sin_, tin_, BD1, b1, s1, t1, BD2, b2, s2, t2,
          Wx, bx, Wxd1, bxd1, Wxd2, bxd2,
          Wdf, Wde, WpTf, WpTe, bdf, bde, ssep, tsep)


def _run_layer(rep, pts, fts, p, K, D, cin, cout, dm, interpret=False):
  B, P, _ = rep.shape
  N = pts.shape[1]
  cmid = cout // 4
  cxin = cout // 2
  weights = _prep_layer(p, K, dm, cin, cout)
  ptsT = jnp.swapaxes(pts, 1, 2)

  body = functools.partial(_layer_body, K, D, P, N, cmid, cxin, cout, dm)

  def full(a):
    nd = a.ndim
    return pl.BlockSpec(a.shape, lambda b, _n=nd: (0,) * _n)

  in_specs = [
      pl.BlockSpec((1, P, 3), lambda b: (b, 0, 0)),
      pl.BlockSpec((1, N, 3), lambda b: (b, 0, 0)),
      pl.BlockSpec((1, 3, N), lambda b: (b, 0, 0)),
      pl.BlockSpec((1, N, cin), lambda b: (b, 0, 0)),
  ] + [full(w) for w in weights]

  out = pl.pallas_call(
      body,
      grid=(B,),
      in_specs=in_specs,
      out_specs=pl.BlockSpec((1, P, cout), lambda b: (b, 0, 0)),
      out_shape=jax.ShapeDtypeStruct((B, P, cout), jnp.float32),
      compiler_params=pltpu.CompilerParams(
          dimension_semantics=("arbitrary",)),
      interpret=interpret,
  )(rep, pts, ptsT, fts, *weights)
  return out


def _idx_body(K, D, P, N, rep_ref, ptsT_ref, out_ref):
  rep = rep_ref[0]
  ptsT = ptsT_ref[0]
  r2 = jnp.sum(rep * rep, axis=1, keepdims=True)
  p2 = jnp.sum(ptsT * ptsT, axis=0, keepdims=True)
  d2 = r2 - 2.0 * _mmbf(rep, ptsT) + p2
  iota = jax.lax.broadcasted_iota(jnp.int32, (P, N), 1)
  iotaf = iota.astype(jnp.float32)
  d2 = _discard(d2, iota, N)
  cols = []
  for kk in range(K):
    oh, d2 = _extract(d2, iota, N)
    cols.append(jnp.sum(oh * iotaf, axis=1, keepdims=True))
    if kk < K - 1 and D > 1:
      for _ in range(D - 1):
        d2 = _discard(d2, iota, N)
  out_ref[0] = jnp.concatenate(cols, axis=1)


def _run_layer_idx(rep, pts, fts, p, K, D, cin, cout, dm, interpret=False):
  B, P, _ = rep.shape
  N = pts.shape[1]
  ptsT = jnp.swapaxes(pts, 1, 2)
  body = functools.partial(_idx_body, K, D, P, N)
  return pl.pallas_call(
      body,
      grid=(B,),
      in_specs=[
          pl.BlockSpec((1, P, 3), lambda b: (b, 0, 0)),
          pl.BlockSpec((1, 3, N), lambda b: (b, 0, 0)),
      ],
      out_specs=pl.BlockSpec((1, P, K), lambda b: (b, 0, 0)),
      out_shape=jax.ShapeDtypeStruct((B, P, K), jnp.float32),
      interpret=interpret,
  )(rep, ptsT)


def _head_body(B, P, f_ref, M_ref, W1, b1, s1, t1, W2, b2, s2, t2, W3, b3,
               out_ref):
  h = jax.nn.relu(_mmbf(f_ref[...], W1[...]) + b1[...]) * s1[...] + t1[...]
  h = jax.nn.relu(_mmbf(h, W2[...]) + b2[...]) * s2[...] + t2[...]
  h = _mmbf(h, W3[...]) + b3[...]
  out_ref[...] = _mm(M_ref[...], h)


def _run_head(f, params, interpret=False):
  B, P, C = f.shape
  fr = f.reshape(B * P, C)
  W1, b1, s1, t1 = _prep_dense(params['fc1'])
  W2, b2, s2, t2 = _prep_dense(params['fc2'])
  W3, b3, _, _ = _prep_dense(params['fc3'])
  M = jnp.kron(jnp.eye(B, dtype=jnp.float32),
               jnp.full((1, P), 1.0 / P, jnp.float32))
  body = functools.partial(_head_body, B, P)
  return pl.pallas_call(
      body,
      out_shape=jax.ShapeDtypeStruct((B, 40), jnp.float32),
      interpret=interpret,
  )(fr, M, W1, b1, s1, t1, W2, b2, s2, t2, W3, b3)


def kernel(x, params, interpret=False):
  pts = jnp.swapaxes(x, 1, 2)
  f = _run_layer(pts, pts, pts, params['l1'], 8, 1, 3, 32, 4, interpret)
  f = _run_layer(pts, pts, f, params['l2'], 8, 2, 32, 64, 2, interpret)
  f = _run_layer(pts, pts, f, params['l3'], 8, 4, 64, 96, 2, interpret)
  rep4 = jnp.take(pts, SAMP4_IDX, axis=1)
  f = _run_layer(rep4, pts, f, params['l4'], 12, 4, 96, 128, 2, interpret)
  f = _run_layer(rep4, rep4, f, params['l5'], 12, 6, 128, 160, 2, interpret)
  return _run_head(f, params, interpret)


# speedup vs baseline: 3.1315x; 1.2218x over previous
import functools

import jax
import jax.numpy as jnp
import numpy as np
from jax.experimental import pallas as pl
from jax.experimental.pallas import tpu as pltpu

EPS5 = 1e-5
RS = 1.0 / np.sqrt(1.0 + EPS5)
BIGF = 3.0e38
SAMP4_IDX = jnp.asarray(
    np.random.default_rng(42).choice(1024, size=120, replace=False),
    dtype=jnp.int32)


def _discard(d2, iota, n):
  idx = jnp.argmin(d2, axis=1)[:, None]
  return jnp.where(iota == idx, BIGF, d2)


def _extract(d2, iota, n):
  idx = jnp.argmin(d2, axis=1)[:, None]
  oh = iota == idx
  return oh.astype(jnp.float32), jnp.where(oh, BIGF, d2)


def _mm(a, b):
  return jax.lax.dot(a, b, precision=jax.lax.Precision.HIGHEST,
                     preferred_element_type=jnp.float32)


def _mmbf(a, b):
  return jax.lax.dot(a.astype(jnp.bfloat16), b.astype(jnp.bfloat16),
                     preferred_element_type=jnp.float32)


def _bf(a):
  return a.astype(jnp.bfloat16).astype(jnp.float32)


def _layer_body(K, D, P, N, cmid, cxin, cout, dm,
                rep_ref, pts_ref, ptsT_ref, fts_ref,
                WinT, bin_, sin_, tin_,
                BD1, b1, s1, t1, BD2, b2, s2, t2,
                Wx, bx, Wxd1, bxd1, Wxd2, bxd2,
                Wdf, Wde, WpTf, WpTe, bdf, bde, ssep, tsep,
                out_ref):
  rep = rep_ref[0]
  pts = pts_ref[0]
  ptsT = ptsT_ref[0]
  fts = fts_ref[0]

  fd = jax.nn.relu(_mmbf(fts, WinT[...]) + bin_[...]) * sin_[...] + tin_[...]

  r2 = jnp.sum(rep * rep, axis=1, keepdims=True)
  p2 = jnp.sum(ptsT * ptsT, axis=0, keepdims=True)
  d2 = r2 - 2.0 * _mmbf(rep, ptsT) + p2

  iota = jax.lax.broadcasted_iota(jnp.int32, (P, N), 1)

  tab = jnp.concatenate([pts, fd], axis=1)
  d2 = _discard(d2, iota, N)
  ptsg = []
  ftsg = []
  for kk in range(K):
    oh, d2 = _extract(d2, iota, N)
    g = _mm(oh, tab)
    ptsg.append(g[:, :3])
    ftsg.append(g[:, 3:])
    if kk < K - 1 and D > 1:
      if D <= 3:
        for _ in range(D - 1):
          d2 = _discard(d2, iota, N)
      else:
        d2 = jax.lax.fori_loop(
            0, D - 1, lambda i, a: _discard(a, iota, N), d2)

  pl_flat = jnp.concatenate(ptsg, axis=1) - jnp.concatenate([rep] * K, axis=1)

  f = jax.nn.relu(_mmbf(pl_flat, BD1[...]) + b1[...]) * s1[...] + t1[...]
  f = jax.nn.relu(_mmbf(f, BD2[...]) + b2[...]) * s2[...] + t2[...]

  X = jax.nn.relu(_mmbf(pl_flat, Wx[...]) + bx[...])
  X = jax.nn.relu(_mmbf(X, Wxd1[...]) + bxd1[...])
  X = _mmbf(X, Wxd2[...]) + bxd2[...]

  Xbf = _bf(X)
  fbf = _bf(f)
  ftsgbf = [_bf(t) for t in ftsg]
  Wdf_ = _bf(Wdf[...])
  Wde_ = _bf(Wde[...])
  bdf_ = bdf[...]
  bde_ = bde[...]
  acc = jnp.zeros((P, cout), jnp.float32)
  for m in range(dm):
    ymf = jnp.zeros((P, cmid), jnp.float32)
    yme = jnp.zeros((P, cxin), jnp.float32)
    for i in range(K):
      fXf = jnp.zeros((P, cmid), jnp.float32)
      fXe = jnp.zeros((P, cxin), jnp.float32)
      for j in range(K):
        xij = Xbf[:, i * K + j:i * K + j + 1]
        fXf = fXf + xij * fbf[:, j * cmid:(j + 1) * cmid]
        fXe = fXe + xij * ftsgbf[j]
      ymf = ymf + _bf(fXf) * Wdf_[m * K + i:m * K + i + 1, :]
      yme = yme + _bf(fXe) * Wde_[m * K + i:m * K + i + 1, :]
    acc = acc + _mmbf(ymf + bdf_[m:m + 1, :], WpTf[...][m * cmid:(m + 1) * cmid, :])
    acc = acc + _mmbf(yme + bde_[m:m + 1, :], WpTe[...][m * cxin:(m + 1) * cxin, :])
  out = jax.nn.relu(acc) * ssep[...] + tsep[...]
  out_ref[0] = out


def _row(v):
  return v.reshape(1, -1)


def _prep_dense(p):
  if 'gamma' in p:
    return p['W'].T, _row(p['b']), _row(p['gamma'] * RS), _row(p['beta'])
  return p['W'].T, _row(p['b']), None, None


def _prep_layer(p, K, dm, cin, cout):
  cmid = cout // 4
  cxin = cout // 2
  xc = p['xconv']
  WinT, bin_, sin_, tin_ = _prep_dense(p['dense_in'])
  eyeK = jnp.eye(K, dtype=jnp.float32)
  BD1 = jnp.kron(eyeK, xc['d1']['W'].T)
  b1 = _row(jnp.tile(xc['d1']['b'], K))
  s1 = _row(jnp.tile(xc['d1']['gamma'] * RS, K))
  t1 = _row(jnp.tile(xc['d1']['beta'], K))
  BD2 = jnp.kron(eyeK, xc['d2']['W'].T)
  b2 = _row(jnp.tile(xc['d2']['b'], K))
  s2 = _row(jnp.tile(xc['d2']['gamma'] * RS, K))
  t2 = _row(jnp.tile(xc['d2']['beta'], K))
  Wx = jnp.transpose(xc['xc_W'][:, :, 0, :], (2, 1, 0)).reshape(3 * K, K * K)
  bx = _row(xc['xc_b'])
  Wxd1 = xc['xd1']['W'].T
  bxd1 = _row(xc['xd1']['b'])
  Wxd2 = xc['xd2']['W'].T
  bxd2 = _row(xc['xd2']['b'])
  sep = xc['sep']
  cc = cmid + cxin
  Wd3 = sep['Wd'][:, 0, 0, :].reshape(cc, dm, K)
  Wdf = jnp.transpose(Wd3[:cmid], (1, 2, 0)).reshape(dm * K, cmid)
  Wde = jnp.transpose(Wd3[cmid:], (1, 2, 0)).reshape(dm * K, cxin)
  Wp2 = sep['Wp'][:, :, 0, 0]
  Wp3 = Wp2.reshape(cout, cc, dm)
  WpTf = jnp.transpose(Wp3[:, :cmid, :], (2, 1, 0)).reshape(dm * cmid, cout)
  WpTe = jnp.transpose(Wp3[:, cmid:, :], (2, 1, 0)).reshape(dm * cxin, cout)
  bd2 = sep['bd'].reshape(cc, dm)
  bdf = bd2[:cmid].T
  bde = bd2[cmid:].T
  ssep = _row(sep['gamma'] * RS)
  tsep = _row(sep['beta'])
  return (WinT, bin_, sin_, tin_, BD1, b1, s1, t1, BD2, b2, s2, t2,
          Wx, bx, Wxd1, bxd1, Wxd2, bxd2,
          Wdf, Wde, WpTf, WpTe, bdf, bde, ssep, tsep)


def _run_layer(rep, pts, fts, p, K, D, cin, cout, dm, interpret=False):
  B, P, _ = rep.shape
  N = pts.shape[1]
  cmid = cout // 4
  cxin = cout // 2
  weights = _prep_layer(p, K, dm, cin, cout)
  ptsT = jnp.swapaxes(pts, 1, 2)

  body = functools.partial(_layer_body, K, D, P, N, cmid, cxin, cout, dm)

  def full(a):
    nd = a.ndim
    return pl.BlockSpec(a.shape, lambda b, _n=nd: (0,) * _n)

  in_specs = [
      pl.BlockSpec((1, P, 3), lambda b: (b, 0, 0)),
      pl.BlockSpec((1, N, 3), lambda b: (b, 0, 0)),
      pl.BlockSpec((1, 3, N), lambda b: (b, 0, 0)),
      pl.BlockSpec((1, N, cin), lambda b: (b, 0, 0)),
  ] + [full(w) for w in weights]

  out = pl.pallas_call(
      body,
      grid=(B,),
      in_specs=in_specs,
      out_specs=pl.BlockSpec((1, P, cout), lambda b: (b, 0, 0)),
      out_shape=jax.ShapeDtypeStruct((B, P, cout), jnp.float32),
      compiler_params=pltpu.CompilerParams(
          dimension_semantics=("parallel",)),
      interpret=interpret,
  )(rep, pts, ptsT, fts, *weights)
  return out


def _idx_body(K, D, P, N, rep_ref, ptsT_ref, out_ref):
  rep = rep_ref[0]
  ptsT = ptsT_ref[0]
  r2 = jnp.sum(rep * rep, axis=1, keepdims=True)
  p2 = jnp.sum(ptsT * ptsT, axis=0, keepdims=True)
  d2 = r2 - 2.0 * _mmbf(rep, ptsT) + p2
  iota = jax.lax.broadcasted_iota(jnp.int32, (P, N), 1)
  iotaf = iota.astype(jnp.float32)
  d2 = _discard(d2, iota, N)
  cols = []
  for kk in range(K):
    oh, d2 = _extract(d2, iota, N)
    cols.append(jnp.sum(oh * iotaf, axis=1, keepdims=True))
    if kk < K - 1 and D > 1:
      for _ in range(D - 1):
        d2 = _discard(d2, iota, N)
  out_ref[0] = jnp.concatenate(cols, axis=1)


def _run_layer_idx(rep, pts, fts, p, K, D, cin, cout, dm, interpret=False):
  B, P, _ = rep.shape
  N = pts.shape[1]
  ptsT = jnp.swapaxes(pts, 1, 2)
  body = functools.partial(_idx_body, K, D, P, N)
  return pl.pallas_call(
      body,
      grid=(B,),
      in_specs=[
          pl.BlockSpec((1, P, 3), lambda b: (b, 0, 0)),
          pl.BlockSpec((1, 3, N), lambda b: (b, 0, 0)),
      ],
      out_specs=pl.BlockSpec((1, P, K), lambda b: (b, 0, 0)),
      out_shape=jax.ShapeDtypeStruct((B, P, K), jnp.float32),
      interpret=interpret,
  )(rep, ptsT)


def _head_body(B, P, f_ref, M_ref, W1, b1, s1, t1, W2, b2, s2, t2, W3, b3,
               out_ref):
  h = jax.nn.relu(_mmbf(f_ref[...], W1[...]) + b1[...]) * s1[...] + t1[...]
  h = jax.nn.relu(_mmbf(h, W2[...]) + b2[...]) * s2[...] + t2[...]
  h = _mmbf(h, W3[...]) + b3[...]
  out_ref[...] = _mm(M_ref[...], h)


def _run_head(f, params, interpret=False):
  B, P, C = f.shape
  fr = f.reshape(B * P, C)
  W1, b1, s1, t1 = _prep_dense(params['fc1'])
  W2, b2, s2, t2 = _prep_dense(params['fc2'])
  W3, b3, _, _ = _prep_dense(params['fc3'])
  M = jnp.kron(jnp.eye(B, dtype=jnp.float32),
               jnp.full((1, P), 1.0 / P, jnp.float32))
  body = functools.partial(_head_body, B, P)
  return pl.pallas_call(
      body,
      out_shape=jax.ShapeDtypeStruct((B, 40), jnp.float32),
      interpret=interpret,
  )(fr, M, W1, b1, s1, t1, W2, b2, s2, t2, W3, b3)


def kernel(x, params, interpret=False):
  pts = jnp.swapaxes(x, 1, 2)
  f = _run_layer(pts, pts, pts, params['l1'], 8, 1, 3, 32, 4, interpret)
  f = _run_layer(pts, pts, f, params['l2'], 8, 2, 32, 64, 2, interpret)
  f = _run_layer(pts, pts, f, params['l3'], 8, 4, 64, 96, 2, interpret)
  rep4 = jnp.take(pts, SAMP4_IDX, axis=1)
  f = _run_layer(rep4, pts, f, params['l4'], 12, 4, 96, 128, 2, interpret)
  f = _run_layer(rep4, rep4, f, params['l5'], 12, 6, 128, 160, 2, interpret)
  return _run_head(f, params, interpret)
